# baseline (speedup 1.0000x reference)
import functools

import jax
import jax.numpy as jnp
from jax import lax
from jax.experimental import pallas as pl
from jax.experimental.pallas import tpu as pltpu
from jax.experimental.pallas import tpu_sc as plsc

B, S, D = 4, 8192, 4096
NUM_CORES = 2
NUM_SUBCORES = 16
NUM_WORKERS = NUM_CORES * NUM_SUBCORES
CHUNK = (B * D) // NUM_WORKERS
CHUNKS_PER_BATCH = D // CHUNK


@functools.partial(
    pl.kernel,
    out_type=jax.ShapeDtypeStruct((B, 1, D), jnp.float32),
    mesh=plsc.VectorSubcoreMesh(core_axis_name="c", subcore_axis_name="s"),
    scratch_types=[pltpu.VMEM((CHUNK,), jnp.float32)],
)
def _last_pool_sc(hid_hbm, out_hbm, buf):
    wid = lax.axis_index("s") * NUM_CORES + lax.axis_index("c")
    b = wid // CHUNKS_PER_BATCH
    off = (wid % CHUNKS_PER_BATCH) * CHUNK
    pltpu.sync_copy(hid_hbm.at[b, 0, pl.ds(off, CHUNK)], buf)
    pltpu.sync_copy(buf, out_hbm.at[b, 0, pl.ds(off, CHUNK)])


def kernel(hidden_state):
    return _last_pool_sc(hidden_state)
